# baseline (speedup 1.0000x reference)
import numpy as np
import jax
import jax.numpy as jnp
from jax import lax
from jax.experimental import pallas as pl
from jax.experimental.pallas import tpu as pltpu

_F32 = jnp.float32
_TN = (((0,), (0,)), ((), ()))
_NT = (((1,), (1,)), ((), ()))
_PK = 5
_BS = 2


def _row_softmax(l):
    m = jnp.max(l, axis=-1, keepdims=True)
    p = jnp.exp(l - m)
    return p / jnp.sum(p, axis=-1, keepdims=True)


def _leaky(x):
    return jnp.where(x >= 0, x, 0.05 * x)


def _consts(T, V):
    G = T // _PK
    VP = _PK * V
    sel_flat = np.tile(np.eye(V, dtype=np.float32) / T, (T, 1))
    blk = np.zeros((128, V), dtype=np.float32)
    for l in range(VP):
        blk[l, l % V] = 1.0 / T
    sel_pack = np.tile(blk, (G, 1))
    spread = (blk > 0).astype(np.float32)
    li = np.arange(128)
    kmask = ((li[:, None] // V) == (li[None, :] // V)).astype(np.float32)
    kmask[VP:, :] = 0.0
    kmask[:, VP:] = 0.0
    return (jnp.asarray(sel_flat), jnp.asarray(sel_pack), jnp.asarray(spread),
            jnp.asarray(kmask))


def kernel(x, W1, b1, g1, be1, W2, b2, g2, be2, W3, b3):
    B, C0, T, V = x.shape
    O1, O2, O3 = W1.shape[0], W2.shape[0], W3.shape[0]
    TV = T * V
    G = T // _PK
    VP = _PK * V
    L = G * 128
    n = B * TV
    NB = B // _BS
    sel_flat, sel_pack, spread, kmask = _consts(T, V)
    x2 = x.reshape(B, C0, TV)

    def block_a_fn(s_ref, km_ref, a):
        sa = jnp.dot(s_ref[...], a, preferred_element_type=_F32)
        return lax.dot_general(sa, s_ref[...], _NT,
                               preferred_element_type=_F32) * km_ref[...]

    def graph_fn(s_ref, km_ref, e, c):
        lg = lax.dot_general(e, e, _TN, preferred_element_type=_F32)
        return block_a_fn(s_ref, km_ref,
                          _row_softmax(lg * (1.0 / np.sqrt(c))))

    def bn_affine(st, g_r, be_r):
        mean = st[:, 0:1] * (1.0 / n)
        var = st[:, 1:2] * (1.0 / n) - mean * mean
        inv = lax.rsqrt(var + 1e-5)
        scale = g_r[...] * inv
        shift = be_r[...] - mean * scale
        return scale, shift

    def body1(x_ref, self_ref, selp_ref, s_ref, km_ref, w1_ref, b1_ref,
              g1_ref, be1_ref, w2_ref, b2_ref, y2_ref, st2_ref, y1s, st1,
              st2s):
        ph = pl.program_id(0)
        b_i = pl.program_id(1)

        @pl.when((ph == 0) & (b_i == 0))
        def _():
            st1[...] = jnp.zeros_like(st1)
            st2s[...] = jnp.zeros_like(st2s)

        @pl.when(ph == 0)
        def _():
            for s in range(_BS):
                xb = x_ref[s]
                e = jnp.dot(xb, self_ref[...], preferred_element_type=_F32)
                blka = graph_fn(s_ref, km_ref, e, C0)
                h = jnp.dot(w1_ref[...], xb, preferred_element_type=_F32)
                h = h + b1_ref[...]
                blka_v = blka[0:VP, :]
                ones = jnp.ones((128, 1), _F32)
                ssum = jnp.zeros((O1, 1), _F32)
                ssq = jnp.zeros((O1, 1), _F32)
                for g in range(G):
                    yg = jnp.dot(h[:, g * VP:(g + 1) * VP], blka_v,
                                 preferred_element_type=_F32)
                    y1s[_BS * b_i + s, :, g * 128:(g + 1) * 128] = yg
                    ssum += jnp.dot(yg, ones, preferred_element_type=_F32)
                    ssq += jnp.dot(yg * yg, ones, preferred_element_type=_F32)
                st1[:, 0:1] += ssum
                st1[:, 1:2] += ssq

        @pl.when(ph == 1)
        def _():
            scale, shift = bn_affine(st1, g1_ref, be1_ref)
            for s in range(_BS):
                z = _leaky(y1s[_BS * b_i + s] * scale + shift)
                e = jnp.dot(z, selp_ref[...], preferred_element_type=_F32)
                blka = graph_fn(s_ref, km_ref, e, O1)
                h = jnp.dot(w2_ref[...], z, preferred_element_type=_F32)
                h = h + b2_ref[...]
                ones = jnp.ones((128, 1), _F32)
                ssum = jnp.zeros((O2, 1), _F32)
                ssq = jnp.zeros((O2, 1), _F32)
                for g in range(G):
                    yg = jnp.dot(h[:, g * 128:(g + 1) * 128], blka,
                                 preferred_element_type=_F32)
                    y2_ref[s, :, g * 128:(g + 1) * 128] = yg
                    ssum += jnp.dot(yg, ones, preferred_element_type=_F32)
                    ssq += jnp.dot(yg * yg, ones, preferred_element_type=_F32)
                st2s[:, 0:1] += ssum
                st2s[:, 1:2] += ssq

            @pl.when(b_i == NB - 1)
            def _():
                st2_ref[...] = st2s[...]

    last = NB - 1
    Y2, S2 = pl.pallas_call(
        body1,
        grid=(2, NB),
        in_specs=[
            pl.BlockSpec((_BS, C0, TV),
                         lambda ph, b: (jnp.where(ph == 0, b, last), 0, 0)),
            pl.BlockSpec((TV, V), lambda ph, b: (0, 0)),
            pl.BlockSpec((L, V), lambda ph, b: (0, 0)),
            pl.BlockSpec(spread.shape, lambda ph, b: (0, 0)),
            pl.BlockSpec(kmask.shape, lambda ph, b: (0, 0)),
            pl.BlockSpec((O1, C0), lambda ph, b: (0, 0)),
            pl.BlockSpec((O1, 1), lambda ph, b: (0, 0)),
            pl.BlockSpec((O1, 1), lambda ph, b: (0, 0)),
            pl.BlockSpec((O1, 1), lambda ph, b: (0, 0)),
            pl.BlockSpec((O2, O1), lambda ph, b: (0, 0)),
            pl.BlockSpec((O2, 1), lambda ph, b: (0, 0)),
        ],
        out_specs=[
            pl.BlockSpec((_BS, O2, L),
                         lambda ph, b: (jnp.where(ph == 1, b, 0), 0, 0)),
            pl.BlockSpec((O2, 2), lambda ph, b: (0, 0)),
        ],
        out_shape=[
            jax.ShapeDtypeStruct((B, O2, L), _F32),
            jax.ShapeDtypeStruct((O2, 2), _F32),
        ],
        scratch_shapes=[
            pltpu.VMEM((B, O1, L), _F32),
            pltpu.VMEM((O1, 2), _F32),
            pltpu.VMEM((O2, 2), _F32),
        ],
    )(x2, sel_flat, sel_pack, spread, kmask, W1, b1.reshape(O1, 1),
      g1.reshape(O1, 1), be1.reshape(O1, 1), W2, b2.reshape(O2, 1))

    BS2 = 8

    def body2(y_ref, st_ref, selp_ref, s_ref, km_ref, g2_ref, be2_ref,
              w3_ref, b3_ref, out_ref):
        scale, shift = bn_affine(st_ref, g2_ref, be2_ref)
        for s in range(BS2):
            z = _leaky(y_ref[s] * scale + shift)
            e = jnp.dot(z, selp_ref[...], preferred_element_type=_F32)
            blka = graph_fn(s_ref, km_ref, e, O2)
            h = jnp.dot(w3_ref[...], z, preferred_element_type=_F32)
            h = h + b3_ref[...]
            for g in range(G):
                yg = jnp.dot(h[:, g * 128:(g + 1) * 128], blka,
                             preferred_element_type=_F32)
                out_ref[s, :, g * VP:(g + 1) * VP] = yg[:, 0:VP]

    out = pl.pallas_call(
        body2,
        grid=(B // BS2,),
        in_specs=[
            pl.BlockSpec((BS2, O2, L), lambda b: (b, 0, 0)),
            pl.BlockSpec((O2, 2), lambda b: (0, 0)),
            pl.BlockSpec((L, V), lambda b: (0, 0)),
            pl.BlockSpec(spread.shape, lambda b: (0, 0)),
            pl.BlockSpec(kmask.shape, lambda b: (0, 0)),
            pl.BlockSpec((O2, 1), lambda b: (0, 0)),
            pl.BlockSpec((O2, 1), lambda b: (0, 0)),
            pl.BlockSpec((O3, O2), lambda b: (0, 0)),
            pl.BlockSpec((O3, 1), lambda b: (0, 0)),
        ],
        out_specs=pl.BlockSpec((BS2, O3, TV), lambda b: (b, 0, 0)),
        out_shape=jax.ShapeDtypeStruct((B, O3, TV), _F32),
    )(Y2, S2, sel_pack, spread, kmask, g2.reshape(O2, 1), be2.reshape(O2, 1),
      W3, b3.reshape(O3, 1))
    return out.reshape(B, O3, T, V)


# speedup vs baseline: 1.6094x; 1.6094x over previous
import numpy as np
import jax
import jax.numpy as jnp
from jax import lax
from jax.experimental import pallas as pl
from jax.experimental.pallas import tpu as pltpu

_F32 = jnp.float32
_TN = (((0,), (0,)), ((), ()))
_NT = (((1,), (1,)), ((), ()))
_PK = 5
_BS = 2


def _row_softmax(l):
    m = jnp.max(l, axis=-1, keepdims=True)
    p = jnp.exp(l - m)
    return p / jnp.sum(p, axis=-1, keepdims=True)


def _leaky(x):
    return jnp.where(x >= 0, x, 0.05 * x)


def _consts(T, V):
    G = T // _PK
    VP = _PK * V
    sel_flat = np.tile(np.eye(V, dtype=np.float32) / T, (T, 1))
    blk = np.zeros((128, V), dtype=np.float32)
    for l in range(VP):
        blk[l, l % V] = 1.0 / T
    sel_pack = np.tile(blk, (G, 1))
    spread = (blk > 0).astype(np.float32)
    li = np.arange(128)
    kmask = ((li[:, None] // V) == (li[None, :] // V)).astype(np.float32)
    kmask[VP:, :] = 0.0
    kmask[:, VP:] = 0.0
    return (jnp.asarray(sel_flat), jnp.asarray(sel_pack), jnp.asarray(spread),
            jnp.asarray(kmask))


def kernel(x, W1, b1, g1, be1, W2, b2, g2, be2, W3, b3):
    B, C0, T, V = x.shape
    O1, O2, O3 = W1.shape[0], W2.shape[0], W3.shape[0]
    TV = T * V
    G = T // _PK
    VP = _PK * V
    L = G * 128
    n = B * TV
    NB = B // _BS
    sel_flat, sel_pack, spread, kmask = _consts(T, V)
    x2 = x.reshape(B, C0, TV)

    def block_a_fn(s_ref, km_ref, a):
        sa = jnp.dot(s_ref[...], a, preferred_element_type=_F32)
        return lax.dot_general(sa, s_ref[...], _NT,
                               preferred_element_type=_F32) * km_ref[...]

    def graph_fn(s_ref, km_ref, e, c):
        lg = lax.dot_general(e, e, _TN, preferred_element_type=_F32)
        return block_a_fn(s_ref, km_ref,
                          _row_softmax(lg * (1.0 / np.sqrt(c))))

    def bn_affine(st, g_r, be_r):
        mean = st[:, 0:1] * (1.0 / n)
        var = st[:, 1:2] * (1.0 / n) - mean * mean
        inv = lax.rsqrt(var + 1e-5)
        scale = g_r[...] * inv
        shift = be_r[...] - mean * scale
        return scale, shift

    def body1(x_ref, self_ref, selp_ref, s_ref, km_ref, w1_ref, b1_ref,
              g1_ref, be1_ref, w2_ref, b2_ref, y2_ref, st2_ref, y1s, st1,
              st2s):
        ph = pl.program_id(0)
        b_i = pl.program_id(1)

        @pl.when((ph == 0) & (b_i == 0))
        def _():
            st1[...] = jnp.zeros_like(st1)
            st2s[...] = jnp.zeros_like(st2s)

        @pl.when(ph == 0)
        def _():
            for s in range(_BS):
                xb = x_ref[s]
                e = jnp.dot(xb, self_ref[...], preferred_element_type=_F32)
                blka = graph_fn(s_ref, km_ref, e, C0)
                h = jnp.dot(w1_ref[...], xb, preferred_element_type=_F32)
                h = h + b1_ref[...]
                blka_v = blka[0:VP, :]
                hst = jnp.concatenate(
                    [h[:, g * VP:(g + 1) * VP] for g in range(G)], axis=0)
                yst = jnp.dot(hst, blka_v,
                              preferred_element_type=_F32)
                sacc = jnp.zeros((O1, 128), _F32)
                qacc = jnp.zeros((O1, 128), _F32)
                for g in range(G):
                    yg = yst[g * O1:(g + 1) * O1]
                    y1s[_BS * b_i + s, :, g * 128:(g + 1) * 128] = yg
                    sacc += yg
                    qacc += yg * yg
                st1[:, 0:1] += jnp.sum(sacc, axis=1, keepdims=True)
                st1[:, 1:2] += jnp.sum(qacc, axis=1, keepdims=True)

        @pl.when(ph == 1)
        def _():
            scale, shift = bn_affine(st1, g1_ref, be1_ref)
            for s in range(_BS):
                z = _leaky(y1s[_BS * b_i + s] * scale + shift)
                e = jnp.dot(z, selp_ref[...], preferred_element_type=_F32)
                blka = graph_fn(s_ref, km_ref, e, O1)
                h = jnp.dot(w2_ref[...], z, preferred_element_type=_F32)
                h = h + b2_ref[...]
                hst = jnp.concatenate(
                    [h[:, g * 128:(g + 1) * 128] for g in range(G)], axis=0)
                yst = jnp.dot(hst, blka,
                              preferred_element_type=_F32)
                sacc = jnp.zeros((O2, 128), _F32)
                qacc = jnp.zeros((O2, 128), _F32)
                for g in range(G):
                    yg = yst[g * O2:(g + 1) * O2]
                    y2_ref[s, :, g * 128:(g + 1) * 128] = yg
                    sacc += yg
                    qacc += yg * yg
                st2s[:, 0:1] += jnp.sum(sacc, axis=1, keepdims=True)
                st2s[:, 1:2] += jnp.sum(qacc, axis=1, keepdims=True)

            @pl.when(b_i == NB - 1)
            def _():
                st2_ref[...] = st2s[...]

    last = NB - 1
    Y2, S2 = pl.pallas_call(
        body1,
        grid=(2, NB),
        in_specs=[
            pl.BlockSpec((_BS, C0, TV),
                         lambda ph, b: (jnp.where(ph == 0, b, last), 0, 0)),
            pl.BlockSpec((TV, V), lambda ph, b: (0, 0)),
            pl.BlockSpec((L, V), lambda ph, b: (0, 0)),
            pl.BlockSpec(spread.shape, lambda ph, b: (0, 0)),
            pl.BlockSpec(kmask.shape, lambda ph, b: (0, 0)),
            pl.BlockSpec((O1, C0), lambda ph, b: (0, 0)),
            pl.BlockSpec((O1, 1), lambda ph, b: (0, 0)),
            pl.BlockSpec((O1, 1), lambda ph, b: (0, 0)),
            pl.BlockSpec((O1, 1), lambda ph, b: (0, 0)),
            pl.BlockSpec((O2, O1), lambda ph, b: (0, 0)),
            pl.BlockSpec((O2, 1), lambda ph, b: (0, 0)),
        ],
        out_specs=[
            pl.BlockSpec((_BS, O2, L),
                         lambda ph, b: (jnp.where(ph == 1, b, 0), 0, 0)),
            pl.BlockSpec((O2, 2), lambda ph, b: (0, 0)),
        ],
        out_shape=[
            jax.ShapeDtypeStruct((B, O2, L), _F32),
            jax.ShapeDtypeStruct((O2, 2), _F32),
        ],
        scratch_shapes=[
            pltpu.VMEM((B, O1, L), _F32),
            pltpu.VMEM((O1, 2), _F32),
            pltpu.VMEM((O2, 2), _F32),
        ],
    )(x2, sel_flat, sel_pack, spread, kmask, W1, b1.reshape(O1, 1),
      g1.reshape(O1, 1), be1.reshape(O1, 1), W2, b2.reshape(O2, 1))

    BS2 = 8

    def body2(y_ref, st_ref, selp_ref, s_ref, km_ref, g2_ref, be2_ref,
              w3_ref, b3_ref, out_ref):
        scale, shift = bn_affine(st_ref, g2_ref, be2_ref)
        for s in range(BS2):
            z = _leaky(y_ref[s] * scale + shift)
            e = jnp.dot(z, selp_ref[...], preferred_element_type=_F32)
            blka = graph_fn(s_ref, km_ref, e, O2)
            h = jnp.dot(w3_ref[...], z, preferred_element_type=_F32)
            h = h + b3_ref[...]
            hst = jnp.concatenate(
                [h[:, g * 128:(g + 1) * 128] for g in range(G)], axis=0)
            yst = jnp.dot(hst, blka,
                          preferred_element_type=_F32)
            for g in range(G):
                out_ref[s, :, g * VP:(g + 1) * VP] = (
                    yst[g * O3:(g + 1) * O3, 0:VP])

    out = pl.pallas_call(
        body2,
        grid=(B // BS2,),
        in_specs=[
            pl.BlockSpec((BS2, O2, L), lambda b: (b, 0, 0)),
            pl.BlockSpec((O2, 2), lambda b: (0, 0)),
            pl.BlockSpec((L, V), lambda b: (0, 0)),
            pl.BlockSpec(spread.shape, lambda b: (0, 0)),
            pl.BlockSpec(kmask.shape, lambda b: (0, 0)),
            pl.BlockSpec((O2, 1), lambda b: (0, 0)),
            pl.BlockSpec((O2, 1), lambda b: (0, 0)),
            pl.BlockSpec((O3, O2), lambda b: (0, 0)),
            pl.BlockSpec((O3, 1), lambda b: (0, 0)),
        ],
        out_specs=pl.BlockSpec((BS2, O3, TV), lambda b: (b, 0, 0)),
        out_shape=jax.ShapeDtypeStruct((B, O3, TV), _F32),
    )(Y2, S2, sel_pack, spread, kmask, g2.reshape(O2, 1), be2.reshape(O2, 1),
      W3, b3.reshape(O3, 1))
    return out.reshape(B, O3, T, V)
